# Initial kernel scaffold; baseline (speedup 1.0000x reference)
import jax
import jax.numpy as jnp
from jax.experimental import pallas as pl


def kernel(sequence, table):
    raise NotImplementedError("write your pallas kernel here")


# speedup vs baseline: 3.6188x; 3.6188x over previous
import functools

import jax
import jax.numpy as jnp
from jax import lax
from jax.experimental import pallas as pl
from jax.experimental.pallas import tpu as pltpu
from jax.experimental.pallas import tpu_sc as plsc


def kernel(sequence, table):
    batch = sequence.shape[0]
    seq_len = sequence.shape[2]
    vocab, dim = table.shape

    mesh = plsc.VectorSubcoreMesh(core_axis_name="c", subcore_axis_name="s")
    num_workers = mesh.num_cores * mesh.num_subcores

    assert seq_len % num_workers == 0
    rows_per_worker = seq_len // num_workers
    chunk = min(64, rows_per_worker)
    assert rows_per_worker % chunk == 0
    steps = rows_per_worker // chunk

    @functools.partial(
        pl.kernel,
        out_type=jax.ShapeDtypeStruct((batch, seq_len, dim), table.dtype),
        mesh=mesh,
        scratch_types=[pltpu.VMEM((chunk, dim), table.dtype)],
    )
    def body(table_hbm, out_hbm, buf):
        wid = lax.axis_index("s") * mesh.num_cores + lax.axis_index("c")
        row0 = wid * rows_per_worker
        for step in range(steps):
            base = row0 + step * chunk
            pltpu.sync_copy(table_hbm.at[pl.ds(base, chunk)], buf)
            for b in range(batch):
                pltpu.sync_copy(buf, out_hbm.at[b, pl.ds(base, chunk)])

    return body(table)
